# baseline (speedup 1.0000x reference)
import jax
import jax.numpy as jnp
from jax import lax
from jax.experimental import pallas as pl
from jax.experimental.pallas import tpu as pltpu
from jax.experimental.pallas import tpu_sc as plsc

BAG = 1024
SEQ = 200
WORD_DIM = 64
POS_DIM = 16
OUT_DIM = WORD_DIM + 2 * POS_DIM
POS_VOCAB = 512
NUM_CORES = 2
NUM_SUBCORES = 16
NW = NUM_CORES * NUM_SUBCORES
CHUNK = 128
KPB = BAG // CHUNK
N_CHUNKS_ALL = SEQ * KPB
PER_W = N_CHUNKS_ALL // NW
IDX_ROWS = 7
NSLOT = 4
GDIST = 2
TOTAL_STEPS = PER_W + GDIST
GROUPS = CHUNK // 16


def _body(wordT, pos1T, pos2T, ww_hbm, wp1T, wp2T, outT,
          widx, p1idx, p2idx, wrows, obuf, p1vm, p2vm, gsems, wsems):
    wid = lax.axis_index("s") * NUM_CORES + lax.axis_index("c")
    c0 = wid * PER_W
    tlo = lax.div(c0, KPB)

    pltpu.sync_copy(wp1T, p1vm)
    pltpu.sync_copy(wp2T, p2vm)
    pltpu.sync_copy(wordT.at[pl.ds(tlo, IDX_ROWS)], widx)
    pltpu.sync_copy(pos1T.at[pl.ds(tlo, IDX_ROWS)], p1idx)
    pltpu.sync_copy(pos2T.at[pl.ds(tlo, IDX_ROWS)], p2idx)

    iota = lax.iota(jnp.int32, 16)

    def chunk_coords(cl):
        cg = c0 + cl
        t = lax.div(cg, KPB)
        k = lax.rem(cg, KPB)
        return t - tlo, t, k

    def word_idx_ref(cl):
        tr, _, k = chunk_coords(cl)
        return widx.at[tr, pl.ds(k * CHUNK, CHUNK)]

    def gather_descr(cl, slot):
        return pltpu.make_async_copy(
            ww_hbm.at[word_idx_ref(cl)],
            wrows.at[pl.ds(slot * CHUNK, CHUNK)],
            gsems.at[slot])

    def write_descr(cl, slot):
        _, t, k = chunk_coords(cl)
        return pltpu.make_async_copy(
            obuf.at[pl.ds(slot * OUT_DIM, OUT_DIM)],
            outT.at[t, pl.ds(0, OUT_DIM), pl.ds(k * CHUNK, CHUNK)],
            wsems.at[slot])

    def transpose_block(cl, slot):
        tr, _, k = chunk_coords(cl)
        orow = slot * OUT_DIM
        wrow = slot * CHUNK
        for g in range(GROUPS):
            lanes = pl.ds(g * 16, 16)
            rvec = iota + (wrow + g * 16)
            for f in range(WORD_DIM):
                v = plsc.load_gather(wrows, [rvec, jnp.full((16,), f, jnp.int32)])
                obuf[orow + f, lanes] = v
            i1 = p1idx[tr, pl.ds(k * CHUNK + g * 16, 16)]
            i2 = p2idx[tr, pl.ds(k * CHUNK + g * 16, 16)]
            for f in range(POS_DIM):
                fvec = jnp.full((16,), f, jnp.int32)
                obuf[orow + WORD_DIM + f, lanes] = plsc.load_gather(p1vm, [fvec, i1])
                obuf[orow + WORD_DIM + POS_DIM + f, lanes] = plsc.load_gather(p2vm, [fvec, i2])

    @pl.loop(0, TOTAL_STEPS)
    def _step(c):
        cd = c - GDIST
        slot = lax.rem(c, NSLOT)
        slotd = lax.rem(cd + NSLOT, NSLOT)

        @pl.when(cd >= 0)
        def _drain():
            gather_descr(cd, slotd).wait()
            transpose_block(cd, slotd)
            write_descr(cd, slotd).start()

        @pl.when(c < PER_W)
        def _fire():
            @pl.when(c >= NSLOT)
            def _wait_prev_write():
                write_descr(c - NSLOT, slot).wait()

            gather_descr(c, slot).start()

    for cl in range(PER_W - NSLOT, PER_W):
        write_descr(cl, cl % NSLOT).wait()


_embed = pl.kernel(
    _body,
    out_type=jax.ShapeDtypeStruct((SEQ, OUT_DIM, BAG), jnp.float32),
    mesh=plsc.VectorSubcoreMesh(core_axis_name="c", subcore_axis_name="s"),
    scratch_types=[
        pltpu.VMEM((IDX_ROWS, BAG), jnp.int32),
        pltpu.VMEM((IDX_ROWS, BAG), jnp.int32),
        pltpu.VMEM((IDX_ROWS, BAG), jnp.int32),
        pltpu.VMEM((NSLOT * CHUNK, WORD_DIM), jnp.float32),
        pltpu.VMEM((NSLOT * OUT_DIM, CHUNK), jnp.float32),
        pltpu.VMEM((POS_DIM, POS_VOCAB), jnp.float32),
        pltpu.VMEM((POS_DIM, POS_VOCAB), jnp.float32),
        pltpu.SemaphoreType.DMA((NSLOT,)),
        pltpu.SemaphoreType.DMA((NSLOT,)),
    ],
    compiler_params=pltpu.CompilerParams(use_tc_tiling_on_sc=False,
                                         needs_layout_passes=False),
)


def kernel(word, position1, position2, W_word, W_pos1, W_pos2):
    outT = _embed(word.T, position1.T, position2.T, W_word, W_pos1.T, W_pos2.T)
    return outT.transpose(2, 0, 1)[:, None, :, :]


# speedup vs baseline: 1.2336x; 1.2336x over previous
import jax
import jax.numpy as jnp
from jax import lax
from jax.experimental import pallas as pl
from jax.experimental.pallas import tpu as pltpu
from jax.experimental.pallas import tpu_sc as plsc

BAG = 1024
SEQ = 200
WORD_DIM = 64
POS_DIM = 16
OUT_DIM = WORD_DIM + 2 * POS_DIM
NUM_CORES = 2
NUM_SUBCORES = 16
NW = NUM_CORES * NUM_SUBCORES
CHUNK = 128
KPB = BAG // CHUNK
N_CHUNKS_ALL = SEQ * KPB
PER_W = N_CHUNKS_ALL // NW
IDX_ROWS = 7
NSLOT = 8
GDIST = 4
TOTAL_STEPS = ((PER_W + GDIST + NSLOT - 1) // NSLOT) * NSLOT

P1_OFF = WORD_DIM
P2_OFF = WORD_DIM + POS_DIM


def _body(wordT, pos1T, pos2T, ww_hbm, wp1_hbm, wp2_hbm, out_hbm,
          widx, p1idx, p2idx, wrows, p1rows, p2rows, gsems, wsems):
    wid = lax.axis_index("s") * NUM_CORES + lax.axis_index("c")
    c0 = wid * PER_W
    tlo = lax.div(c0, KPB)

    pltpu.sync_copy(wordT.at[pl.ds(tlo, IDX_ROWS)], widx)
    pltpu.sync_copy(pos1T.at[pl.ds(tlo, IDX_ROWS)], p1idx)
    pltpu.sync_copy(pos2T.at[pl.ds(tlo, IDX_ROWS)], p2idx)

    def coords(cl):
        cg = c0 + cl
        t = lax.div(cg, KPB)
        return t - tlo, t, lax.rem(cg, KPB)

    def slot_bufs(b):
        sl = pl.ds(b * CHUNK, CHUNK)
        return wrows.at[sl], p1rows.at[sl], p2rows.at[sl]

    def out_slices(cl):
        tr, t, k = coords(cl)
        bags = pl.ds(k * CHUNK, CHUNK)
        return (out_hbm.at[bags, 0, t, pl.ds(0, WORD_DIM)],
                out_hbm.at[bags, 0, t, pl.ds(P1_OFF, POS_DIM)],
                out_hbm.at[bags, 0, t, pl.ds(P2_OFF, POS_DIM)])

    def idx_slices(cl):
        tr, t, k = coords(cl)
        sl = pl.ds(k * CHUNK, CHUNK)
        return widx.at[tr, sl], p1idx.at[tr, sl], p2idx.at[tr, sl]

    def wait_writes(b, cl):
        wr, p1r, p2r = slot_bufs(b)
        ow, o1, o2 = out_slices(cl)
        pltpu.make_async_copy(wr, ow, wsems.at[b]).wait()
        pltpu.make_async_copy(p1r, o1, wsems.at[b]).wait()
        pltpu.make_async_copy(p2r, o2, wsems.at[b]).wait()

    @pl.loop(0, TOTAL_STEPS, step=NSLOT)
    def _steps(c0_step):
        for b in range(NSLOT):
            c = c0_step + b
            cd = c - GDIST
            bd = (b - GDIST) % NSLOT

            @pl.when((cd >= 0) & (cd < PER_W))
            def _drain():
                wr, p1r, p2r = slot_bufs(bd)
                iw, i1, i2 = idx_slices(cd)
                ow, o1, o2 = out_slices(cd)
                pltpu.make_async_copy(ww_hbm.at[iw], wr, gsems.at[bd]).wait()
                pltpu.make_async_copy(wp1_hbm.at[i1], p1r, gsems.at[bd]).wait()
                pltpu.make_async_copy(wp2_hbm.at[i2], p2r, gsems.at[bd]).wait()
                pltpu.async_copy(wr, ow, wsems.at[bd])
                pltpu.async_copy(p1r, o1, wsems.at[bd])
                pltpu.async_copy(p2r, o2, wsems.at[bd])

            @pl.when(c < PER_W)
            def _fire():
                @pl.when(c >= NSLOT)
                def _wait_prev_write():
                    wait_writes(b, c - NSLOT)

                wr, p1r, p2r = slot_bufs(b)
                iw, i1, i2 = idx_slices(c)
                pltpu.async_copy(ww_hbm.at[iw], wr, gsems.at[b])
                pltpu.async_copy(wp1_hbm.at[i1], p1r, gsems.at[b])
                pltpu.async_copy(wp2_hbm.at[i2], p2r, gsems.at[b])

    for cl in range(PER_W - NSLOT, PER_W):
        wait_writes(cl % NSLOT, cl)


_embed = pl.kernel(
    _body,
    out_type=jax.ShapeDtypeStruct((BAG, 1, SEQ, OUT_DIM), jnp.float32),
    mesh=plsc.VectorSubcoreMesh(core_axis_name="c", subcore_axis_name="s"),
    scratch_types=[
        pltpu.VMEM((IDX_ROWS, BAG), jnp.int32),
        pltpu.VMEM((IDX_ROWS, BAG), jnp.int32),
        pltpu.VMEM((IDX_ROWS, BAG), jnp.int32),
        pltpu.VMEM((NSLOT * CHUNK, WORD_DIM), jnp.float32),
        pltpu.VMEM((NSLOT * CHUNK, POS_DIM), jnp.float32),
        pltpu.VMEM((NSLOT * CHUNK, POS_DIM), jnp.float32),
        pltpu.SemaphoreType.DMA((NSLOT,)),
        pltpu.SemaphoreType.DMA((NSLOT,)),
    ],
    compiler_params=pltpu.CompilerParams(use_tc_tiling_on_sc=False),
)


def kernel(word, position1, position2, W_word, W_pos1, W_pos2):
    return _embed(word.T, position1.T, position2.T, W_word, W_pos1, W_pos2)


# speedup vs baseline: 1.2725x; 1.0315x over previous
import jax
import jax.numpy as jnp
from jax import lax
from jax.experimental import pallas as pl
from jax.experimental.pallas import tpu as pltpu
from jax.experimental.pallas import tpu_sc as plsc

BAG = 1024
SEQ = 200
WORD_DIM = 64
POS_DIM = 16
OUT_DIM = WORD_DIM + 2 * POS_DIM
WPAD = 128
POS_VOCAB = 512
NUM_CORES = 2
NUM_SUBCORES = 16
NW = NUM_CORES * NUM_SUBCORES
CHUNK = 128
KPB = BAG // CHUNK
N_CHUNKS_ALL = SEQ * KPB
PER_W = N_CHUNKS_ALL // NW
IDX_ROWS = 7
NSLOT = 3
GDIST = 2
TOTAL_STEPS = PER_W + GDIST
OPITCH = 129
GROUPS = CHUNK // 16


def _body(wordT, pos1T, pos2T, ww_hbm, wp1T, wp2T, outT,
          widx, p1idx, p2idx, wrows, obuf, p1vm, p2vm, gsems, wsems):
    wid = lax.axis_index("s") * NUM_CORES + lax.axis_index("c")
    c0 = wid * PER_W
    tlo = lax.div(c0, KPB)

    pltpu.sync_copy(wp1T, p1vm)
    pltpu.sync_copy(wp2T, p2vm)
    pltpu.sync_copy(wordT.at[pl.ds(tlo, IDX_ROWS)], widx)
    pltpu.sync_copy(pos1T.at[pl.ds(tlo, IDX_ROWS)], p1idx)
    pltpu.sync_copy(pos2T.at[pl.ds(tlo, IDX_ROWS)], p2idx)

    iota = lax.iota(jnp.int32, 16)
    ones = jnp.full((16,), 1, jnp.int32)

    def coords(cl):
        cg = c0 + cl
        t = lax.div(cg, KPB)
        return t - tlo, t, lax.rem(cg, KPB)

    def gather_descr(cl, slot):
        tr, _, k = coords(cl)
        return pltpu.make_async_copy(
            ww_hbm.at[widx.at[tr, pl.ds(k * CHUNK, CHUNK)]],
            wrows.at[pl.ds(slot * CHUNK, CHUNK)],
            gsems.at[slot])

    def write_descr(cl, slot):
        _, t, k = coords(cl)
        return pltpu.make_async_copy(
            obuf.at[pl.ds(slot * OUT_DIM, OUT_DIM), pl.ds(0, CHUNK)],
            outT.at[t, pl.ds(0, OUT_DIM), pl.ds(k * CHUNK, CHUNK)],
            wsems.at[slot])

    def transpose_block(cl, slot):
        tr, _, k = coords(cl)
        orow = slot * OUT_DIM
        wrow = slot * CHUNK
        rvecs = [iota + (orow + f0) for f0 in (0, 16, 32, 48)]
        cvec = jnp.full((16,), 0, jnp.int32)
        for b in range(CHUNK):
            for j, f0 in enumerate((0, 16, 32, 48)):
                v = wrows[wrow + b, pl.ds(f0, 16)]
                plsc.store_scatter(obuf, [rvecs[j], cvec], v)
            cvec = cvec + ones
        for g in range(GROUPS):
            lanes = pl.ds(g * 16, 16)
            i1 = p1idx[tr, pl.ds(k * CHUNK + g * 16, 16)]
            i2 = p2idx[tr, pl.ds(k * CHUNK + g * 16, 16)]
            for f in range(POS_DIM):
                fvec = jnp.full((16,), f, jnp.int32)
                obuf[orow + WORD_DIM + f, lanes] = plsc.load_gather(p1vm, [fvec, i1])
                obuf[orow + WORD_DIM + POS_DIM + f, lanes] = plsc.load_gather(p2vm, [fvec, i2])

    @pl.loop(0, TOTAL_STEPS)
    def _step(c):
        cd = c - GDIST
        slot = lax.rem(c, NSLOT)
        slotd = lax.rem(cd + NSLOT, NSLOT)

        @pl.when(cd >= 0)
        def _drain():
            gather_descr(cd, slotd).wait()
            transpose_block(cd, slotd)
            write_descr(cd, slotd).start()

        @pl.when(c < PER_W)
        def _fire():
            @pl.when(c >= NSLOT)
            def _wait_prev_write():
                write_descr(c - NSLOT, slot).wait()

            gather_descr(c, slot).start()

    for cl in range(PER_W - NSLOT, PER_W):
        write_descr(cl, cl % NSLOT).wait()


_embed = pl.kernel(
    _body,
    out_type=jax.ShapeDtypeStruct((SEQ, OUT_DIM, BAG), jnp.float32),
    mesh=plsc.VectorSubcoreMesh(core_axis_name="c", subcore_axis_name="s"),
    scratch_types=[
        pltpu.VMEM((IDX_ROWS, BAG), jnp.int32),
        pltpu.VMEM((IDX_ROWS, BAG), jnp.int32),
        pltpu.VMEM((IDX_ROWS, BAG), jnp.int32),
        pltpu.VMEM((NSLOT * CHUNK, WPAD), jnp.float32),
        pltpu.VMEM((NSLOT * OUT_DIM, OPITCH), jnp.float32),
        pltpu.VMEM((POS_DIM, POS_VOCAB), jnp.float32),
        pltpu.VMEM((POS_DIM, POS_VOCAB), jnp.float32),
        pltpu.SemaphoreType.DMA((NSLOT,)),
        pltpu.SemaphoreType.DMA((NSLOT,)),
    ],
    compiler_params=pltpu.CompilerParams(use_tc_tiling_on_sc=False,
                                         needs_layout_passes=False),
)


def kernel(word, position1, position2, W_word, W_pos1, W_pos2):
    w_padded = jnp.pad(W_word, ((0, 0), (0, WPAD - WORD_DIM)))
    outT = _embed(word.T, position1.T, position2.T, w_padded,
                  W_pos1.T, W_pos2.T)
    return outT.transpose(2, 0, 1)[:, None, :, :]


# speedup vs baseline: 1.2914x; 1.0149x over previous
import jax
import jax.numpy as jnp
from jax import lax
from jax.experimental import pallas as pl
from jax.experimental.pallas import tpu as pltpu
from jax.experimental.pallas import tpu_sc as plsc

BAG = 1024
SEQ = 200
WORD_DIM = 64
POS_DIM = 16
OUT_DIM = WORD_DIM + 2 * POS_DIM
NUM_CORES = 2
NUM_SUBCORES = 16
NW = NUM_CORES * NUM_SUBCORES
CHUNK = 128
KPB = BAG // CHUNK
N_CHUNKS_ALL = SEQ * KPB
PER_W = N_CHUNKS_ALL // NW
IDX_ROWS = 7
NSLOT = 4
GDIST = 2
WPAD = 2 * WORD_DIM
TOTAL_STEPS = ((PER_W + GDIST + NSLOT - 1) // NSLOT) * NSLOT

P1_OFF = WORD_DIM
P2_OFF = WORD_DIM + POS_DIM


def _body(wordT, pos1T, pos2T, ww_hbm, wp1_hbm, wp2_hbm, out_hbm,
          widx, p1idx, p2idx, wrows, p1rows, p2rows, gsems, wsems):
    wid = lax.axis_index("s") * NUM_CORES + lax.axis_index("c")
    c0 = wid * PER_W
    tlo = lax.div(c0, KPB)

    pltpu.sync_copy(wordT.at[pl.ds(tlo, IDX_ROWS)], widx)
    pltpu.sync_copy(pos1T.at[pl.ds(tlo, IDX_ROWS)], p1idx)
    pltpu.sync_copy(pos2T.at[pl.ds(tlo, IDX_ROWS)], p2idx)

    def coords(cl):
        cg = c0 + cl
        t = lax.div(cg, KPB)
        return t - tlo, t, lax.rem(cg, KPB)

    def slot_bufs(b):
        sl = pl.ds(b * CHUNK, CHUNK)
        return wrows.at[sl], p1rows.at[sl], p2rows.at[sl]

    def word_src(b):
        return wrows.at[pl.ds(b * CHUNK, CHUNK), pl.ds(0, WORD_DIM)]

    def out_slices(cl):
        tr, t, k = coords(cl)
        bags = pl.ds(k * CHUNK, CHUNK)
        return (out_hbm.at[bags, 0, t, pl.ds(0, WORD_DIM)],
                out_hbm.at[bags, 0, t, pl.ds(P1_OFF, POS_DIM)],
                out_hbm.at[bags, 0, t, pl.ds(P2_OFF, POS_DIM)])

    def idx_slices(cl):
        tr, t, k = coords(cl)
        sl = pl.ds(k * CHUNK, CHUNK)
        return widx.at[tr, sl], p1idx.at[tr, sl], p2idx.at[tr, sl]

    def wait_writes(b, cl):
        _, p1r, p2r = slot_bufs(b)
        ow, o1, o2 = out_slices(cl)
        pltpu.make_async_copy(word_src(b), ow, wsems.at[b]).wait()
        pltpu.make_async_copy(p1r, o1, wsems.at[b]).wait()
        pltpu.make_async_copy(p2r, o2, wsems.at[b]).wait()

    @pl.loop(0, TOTAL_STEPS, step=NSLOT)
    def _steps(c0_step):
        for b in range(NSLOT):
            c = c0_step + b
            cd = c - GDIST
            bd = (b - GDIST) % NSLOT

            @pl.when((cd >= 0) & (cd < PER_W))
            def _drain():
                wr, p1r, p2r = slot_bufs(bd)
                iw, i1, i2 = idx_slices(cd)
                ow, o1, o2 = out_slices(cd)
                pltpu.make_async_copy(ww_hbm.at[iw], wr, gsems.at[bd]).wait()
                pltpu.make_async_copy(wp1_hbm.at[i1], p1r, gsems.at[bd]).wait()
                pltpu.make_async_copy(wp2_hbm.at[i2], p2r, gsems.at[bd]).wait()
                pltpu.async_copy(word_src(bd), ow, wsems.at[bd])
                pltpu.async_copy(p1r, o1, wsems.at[bd])
                pltpu.async_copy(p2r, o2, wsems.at[bd])

            @pl.when(c < PER_W)
            def _fire():
                @pl.when(c >= NSLOT)
                def _wait_prev_write():
                    wait_writes(b, c - NSLOT)

                wr, p1r, p2r = slot_bufs(b)
                iw, i1, i2 = idx_slices(c)
                pltpu.async_copy(ww_hbm.at[iw], wr, gsems.at[b])
                pltpu.async_copy(wp1_hbm.at[i1], p1r, gsems.at[b])
                pltpu.async_copy(wp2_hbm.at[i2], p2r, gsems.at[b])

    for cl in range(PER_W - NSLOT, PER_W):
        wait_writes(cl % NSLOT, cl)


_embed = pl.kernel(
    _body,
    out_type=jax.ShapeDtypeStruct((BAG, 1, SEQ, OUT_DIM), jnp.float32),
    mesh=plsc.VectorSubcoreMesh(core_axis_name="c", subcore_axis_name="s"),
    scratch_types=[
        pltpu.VMEM((IDX_ROWS, BAG), jnp.int32),
        pltpu.VMEM((IDX_ROWS, BAG), jnp.int32),
        pltpu.VMEM((IDX_ROWS, BAG), jnp.int32),
        pltpu.VMEM((NSLOT * CHUNK, WPAD), jnp.float32),
        pltpu.VMEM((NSLOT * CHUNK, POS_DIM), jnp.float32),
        pltpu.VMEM((NSLOT * CHUNK, POS_DIM), jnp.float32),
        pltpu.SemaphoreType.DMA((NSLOT,)),
        pltpu.SemaphoreType.DMA((NSLOT,)),
    ],
    compiler_params=pltpu.CompilerParams(use_tc_tiling_on_sc=False),
)


def kernel(word, position1, position2, W_word, W_pos1, W_pos2):
    w_padded = jnp.pad(W_word, ((0, 0), (0, WPAD - WORD_DIM)))
    return _embed(word.T, position1.T, position2.T, w_padded, W_pos1, W_pos2)
